# baseline (speedup 1.0000x reference)
import functools

import jax
import jax.numpy as jnp
from jax import lax
from jax.experimental import pallas as pl
from jax.experimental.pallas import tpu as pltpu
from jax.experimental.pallas import tpu_sc as plsc

RS = 3.0
RC = 6.0

NC = 2
NS = 16
NW = NC * NS

F32 = jnp.float32
I32 = jnp.int32


def _mesh():
    return plsc.VectorSubcoreMesh(core_axis_name="c", subcore_axis_name="s",
                                  num_cores=NC, num_subcores=NS)


_SC_PARAMS = pltpu.CompilerParams(use_tc_tiling_on_sc=False)
_SC_PARAMS_NL = pltpu.CompilerParams(use_tc_tiling_on_sc=False,
                                     needs_layout_passes=False)


def _make_gather_attr(RB, N, A):
    @functools.partial(
        pl.kernel,
        out_type=(
            jax.ShapeDtypeStruct((RB, 128, A), F32),
            jax.ShapeDtypeStruct((RB, 128, A), F32),
        ),
        mesh=_mesh(),
        compiler_params=_SC_PARAMS,
        scratch_types=[
            pltpu.VMEM((128,), I32),
            pltpu.VMEM((128,), I32),
            pltpu.VMEM((128, A), F32),
            pltpu.VMEM((128, A), F32),
            pltpu.SemaphoreType.DMA,
            pltpu.SemaphoreType.DMA,
        ],
    )
    def gather_attr(idx0_hbm, idx1_hbm, atom_hbm, out0, out1, i0v, i1v, r0, r1,
                    s0, s1):
        c = lax.axis_index("c")
        s = lax.axis_index("s")
        wid = s * NC + c
        nrows = (RB - wid + NW - 1) // NW

        def body(i, carry):
            r = wid + i * NW
            pltpu.sync_copy(idx0_hbm.at[r], i0v)
            pltpu.sync_copy(idx1_hbm.at[r], i1v)
            cp0 = pltpu.async_copy(atom_hbm.at[i0v], r0, s0)
            cp1 = pltpu.async_copy(atom_hbm.at[i1v], r1, s1)
            cp0.wait()
            cp1.wait()
            pltpu.sync_copy(r0, out0.at[r])
            pltpu.sync_copy(r1, out1.at[r])
            return carry

        lax.fori_loop(0, nrows, body, 0)

    return gather_attr


def _snorm_body(x_ref, y_ref, z_ref, out_ref):
    x = x_ref[...]
    y = y_ref[...]
    z = z_ref[...]
    r2 = x * x + y * y + z * z
    r = jnp.sqrt(r2)
    r_safe = jnp.maximum(r, 1e-6)
    inv = 1.0 / r_safe
    t = (r - RC) / (RS - RC)
    poly = t * t * t * (10.0 + t * (-15.0 + 6.0 * t)) + 1.0
    mid = inv * poly
    out_ref[...] = jnp.where(r < RS, inv,
                             jnp.where(r < RC, mid, jnp.zeros_like(r)))


def _run_snorm(envx, envy, envz, RB):
    return pl.pallas_call(
        _snorm_body,
        grid=(1,),
        in_specs=[pl.BlockSpec((RB, 128), lambda i: (0, 0))] * 3,
        out_specs=pl.BlockSpec((RB, 128), lambda i: (0, 0)),
        out_shape=jax.ShapeDtypeStruct((RB, 128), F32),
    )(envx, envy, envz)


def _msg_body(sn_ref, env_ref, a0_ref, a1_ref, w1a_ref, w1b_ref, w1c_ref,
              b1_ref, w2_ref, b2_ref, out_ref):
    snorm = sn_ref[...]
    env = env_ref[...]
    a0 = a0_ref[...]
    a1 = a1_ref[...]

    pre = (jnp.dot(snorm, w1a_ref[...], preferred_element_type=F32) +
           jnp.dot(a0, w1b_ref[...], preferred_element_type=F32) +
           jnp.dot(a1, w1c_ref[...], preferred_element_type=F32) +
           b1_ref[...])
    h1 = jnp.tanh(pre)
    h2 = jnp.dot(h1, w2_ref[...], preferred_element_type=F32) + b2_ref[...]

    col10 = lax.broadcasted_iota(I32, (10, 32), 1)
    row10 = lax.broadcasted_iota(I32, (10, 32), 0)
    rmat = jnp.where((col10 < 30) & (col10 // 3 == row10), 1.0, 0.0)
    col3 = lax.broadcasted_iota(I32, (3, 32), 1)
    row3 = lax.broadcasted_iota(I32, (3, 32), 0)
    cmat = jnp.where((col3 < 30) & (col3 % 3 == row3), 1.0, 0.0)
    B = env.shape[0]
    cnt_col = jnp.where(lax.broadcasted_iota(I32, (B, 32), 1) == 30, 1.0, 0.0)
    out = (jnp.dot(h2, rmat, preferred_element_type=F32) *
           jnp.dot(env, cmat, preferred_element_type=F32)) + cnt_col
    out_ref[...] = out


def _run_msg(snorm, env_vectors, attr0_16, attr1_16, W1, b1, W2, b2, E, BE):
    w1a = W1[0:1, :]
    w1b = jnp.zeros((16, 20), F32).at[0:4].set(W1[1:5])
    w1c = jnp.zeros((16, 20), F32).at[0:4].set(W1[5:9])
    grid = (E // BE,)
    return pl.pallas_call(
        _msg_body,
        grid=grid,
        in_specs=[
            pl.BlockSpec((BE, 1), lambda i: (i, 0)),
            pl.BlockSpec((BE, 3), lambda i: (i, 0)),
            pl.BlockSpec((BE, 16), lambda i: (i, 0)),
            pl.BlockSpec((BE, 16), lambda i: (i, 0)),
            pl.BlockSpec((1, 20), lambda i: (0, 0)),
            pl.BlockSpec((16, 20), lambda i: (0, 0)),
            pl.BlockSpec((16, 20), lambda i: (0, 0)),
            pl.BlockSpec((1, 20), lambda i: (0, 0)),
            pl.BlockSpec((20, 10), lambda i: (0, 0)),
            pl.BlockSpec((1, 10), lambda i: (0, 0)),
        ],
        out_specs=pl.BlockSpec((BE, 32), lambda i: (i, 0)),
        out_shape=jax.ShapeDtypeStruct((E, 32), F32),
    )(snorm, env_vectors, attr0_16, attr1_16, w1a, w1b, w1c,
      b1.reshape(1, 20), W2, b2.reshape(1, 10))


def _make_scatter_msg(RB, N):
    RB_SC = RB // NC
    CH = 5
    NCHUNK = RB_SC // CH
    ZR = 125
    NROW_T = N // NS
    NZ = NROW_T // ZR

    @functools.partial(
        pl.kernel,
        out_type=jax.ShapeDtypeStruct((NC, N, 32), F32),
        mesh=_mesh(),
        compiler_params=_SC_PARAMS,
        scratch_types=[
            pltpu.VMEM_SHARED((N, 32), F32),
            pltpu.VMEM((CH, 128, 32), F32),
            pltpu.VMEM((CH, 128), I32),
            pltpu.VMEM((ZR, 32), F32),
        ],
    )
    def scatter_msg(msg_hbm, dst_hbm, out, accum, mbuf, idxbuf, zbuf):
        c = lax.axis_index("c")
        s = lax.axis_index("s")

        zero16 = jnp.zeros((16,), F32)

        def zrow(i, carry):
            zbuf[i, pl.ds(0, 16)] = zero16
            zbuf[i, pl.ds(16, 16)] = zero16
            return carry

        lax.fori_loop(0, ZR, zrow, 0)

        base = s * NROW_T

        def zcopy(k, carry):
            pltpu.sync_copy(zbuf, accum.at[pl.ds(base + k * ZR, ZR)])
            return carry

        lax.fori_loop(0, NZ, zcopy, 0)
        plsc.subcore_barrier()

        nch = (NCHUNK - s + NS - 1) // NS

        def sbody(k, carry):
            j = s + k * NS
            row0 = c * RB_SC + j * CH
            pltpu.sync_copy(msg_hbm.at[pl.ds(row0, CH)], mbuf)
            pltpu.sync_copy(dst_hbm.at[pl.ds(row0, CH)], idxbuf)
            for jj in range(CH):
                pltpu.sync_copy(mbuf.at[jj], accum.at[idxbuf.at[jj]],
                                add=True)
            return carry

        lax.fori_loop(0, nch, sbody, 0)
        plsc.subcore_barrier()

        def obody(k, carry):
            r0 = base + k * ZR
            pltpu.sync_copy(accum.at[pl.ds(r0, ZR)], out.at[c, pl.ds(r0, ZR)])
            return carry

        lax.fori_loop(0, NZ, obody, 0)

    return scatter_msg


def _gram_body(p_ref, out_ref, pad_ref):
    p = p_ref[...]
    sfull = p[0] + p[1]
    cnt = jnp.maximum(sfull[:, 30:31], 1.0)
    a = sfull[:, :30] / cnt

    j30 = lax.broadcasted_iota(I32, (30, 100), 0)
    m = lax.broadcasted_iota(I32, (30, 100), 1)
    out = None
    for c in range(3):
        m1 = jnp.where(j30 == 3 * (m // 10) + c, 1.0, 0.0)
        m2 = jnp.where(j30 == 3 * (m % 10) + c, 1.0, 0.0)
        term = (jnp.dot(a, m1, preferred_element_type=F32) *
                jnp.dot(a, m2, preferred_element_type=F32))
        out = term if out is None else out + term
    out_ref[...] = out
    B = out.shape[0]
    pad_ref[...] = jnp.concatenate([out, jnp.zeros((B, 12), F32)], axis=1)


def _run_gram(partials, N, BN):
    grid = (N // BN,)
    return pl.pallas_call(
        _gram_body,
        grid=grid,
        in_specs=[pl.BlockSpec((2, BN, 32), lambda i: (0, i, 0))],
        out_specs=(
            pl.BlockSpec((BN, 100), lambda i: (i, 0)),
            pl.BlockSpec((BN, 112), lambda i: (i, 0)),
        ),
        out_shape=(
            jax.ShapeDtypeStruct((N, 100), F32),
            jax.ShapeDtypeStruct((N, 112), F32),
        ),
    )(partials)


def _make_edge_gather(RB, N, P):
    @functools.partial(
        pl.kernel,
        out_type=jax.ShapeDtypeStruct((RB, 12800), F32),
        mesh=_mesh(),
        compiler_params=_SC_PARAMS_NL,
        scratch_types=[
            pltpu.VMEM((128,), I32),
            pltpu.VMEM((128,), I32),
            pltpu.VMEM((128, P), F32),
            pltpu.VMEM((128, P), F32),
            pltpu.VMEM((12800,), F32),
            pltpu.SemaphoreType.DMA,
            pltpu.SemaphoreType.DMA,
        ],
    )
    def edge_gather(nd_hbm, idx0_hbm, idx1_hbm, out, i0v, i1v, r0, r1, ob,
                    s0, s1):
        c = lax.axis_index("c")
        s = lax.axis_index("s")
        wid = s * NC + c
        nrows = (RB - wid + NW - 1) // NW
        lanes = lax.iota(I32, 16)

        def body(i, carry):
            r = wid + i * NW
            pltpu.sync_copy(idx0_hbm.at[r], i0v)
            pltpu.sync_copy(idx1_hbm.at[r], i1v)
            cp0 = pltpu.async_copy(nd_hbm.at[i0v], r0, s0)
            cp1 = pltpu.async_copy(nd_hbm.at[i1v], r1, s1)
            cp0.wait()
            cp1.wait()

            def rep(kk, carry2):
                for u in range(8):
                    k0 = (kk * 8 + u) * 16
                    kv = k0 + lanes
                    iv = lax.shift_right_logical(kv * 41944, 22)
                    jv = kv - iv * 100
                    g0 = plsc.load_gather(r0, [iv, jv])
                    g1 = plsc.load_gather(r1, [iv, jv])
                    ob[pl.ds(k0, 16)] = g0 + g1
                return carry2

            lax.fori_loop(0, 100, rep, 0)

            pltpu.sync_copy(ob, out.at[r])
            return carry

        lax.fori_loop(0, nrows, body, 0)

    return edge_gather


def kernel(env_vectors, atom_attr, W1, b1, W2, b2, env_index, edge_index):
    N = atom_attr.shape[0]
    E = env_vectors.shape[0]
    A = atom_attr.shape[1]
    RB = E // 128
    P = 112
    BE = 8000
    BN = 2000

    ei0 = env_index[0].reshape(RB, 128)
    ei1 = env_index[1].reshape(RB, 128)
    de0 = edge_index[0].reshape(RB, 128)
    de1 = edge_index[1].reshape(RB, 128)

    atom_pad = jnp.pad(atom_attr, ((0, 0), (0, 16 - A)))
    attr0_3d, attr1_3d = _make_gather_attr(RB, N, 16)(ei0, ei1, atom_pad)
    attr0_16 = attr0_3d.reshape(E, 16)
    attr1_16 = attr1_3d.reshape(E, 16)

    envx = env_vectors[:, 0].reshape(RB, 128)
    envy = env_vectors[:, 1].reshape(RB, 128)
    envz = env_vectors[:, 2].reshape(RB, 128)
    snorm = _run_snorm(envx, envy, envz, RB).reshape(E, 1)

    msg = _run_msg(snorm, env_vectors, attr0_16, attr1_16, W1, b1, W2, b2,
                   E, BE)
    msg3d = msg.reshape(RB, 128, 32)

    partials = _make_scatter_msg(RB, N)(msg3d, ei1)

    node_desc, node_pad = _run_gram(partials, N, BN)

    edge2d = _make_edge_gather(RB, N, P)(node_pad, de0, de1)
    edge_desc = edge2d.reshape(E, 100)

    return node_desc, edge_desc


# speedup vs baseline: 1.1313x; 1.1313x over previous
import functools

import jax
import jax.numpy as jnp
from jax import lax
from jax.experimental import pallas as pl
from jax.experimental.pallas import tpu as pltpu
from jax.experimental.pallas import tpu_sc as plsc

RS = 3.0
RC = 6.0

NC = 2
NS = 16
NW = NC * NS

F32 = jnp.float32
I32 = jnp.int32


def _mesh():
    return plsc.VectorSubcoreMesh(core_axis_name="c", subcore_axis_name="s",
                                  num_cores=NC, num_subcores=NS)


_SC_PARAMS = pltpu.CompilerParams(use_tc_tiling_on_sc=False)
_SC_PARAMS_NL = pltpu.CompilerParams(use_tc_tiling_on_sc=False,
                                     needs_layout_passes=False)


def _make_gather_attr(RB, N, A):
    @functools.partial(
        pl.kernel,
        out_type=(
            jax.ShapeDtypeStruct((RB, 128, A), F32),
            jax.ShapeDtypeStruct((RB, 128, A), F32),
        ),
        mesh=_mesh(),
        compiler_params=_SC_PARAMS,
        scratch_types=[
            pltpu.VMEM((128,), I32),
            pltpu.VMEM((128,), I32),
            pltpu.VMEM((128, A), F32),
            pltpu.VMEM((128, A), F32),
            pltpu.SemaphoreType.DMA,
            pltpu.SemaphoreType.DMA,
        ],
    )
    def gather_attr(idx0_hbm, idx1_hbm, atom_hbm, out0, out1, i0v, i1v, r0, r1,
                    s0, s1):
        c = lax.axis_index("c")
        s = lax.axis_index("s")
        wid = s * NC + c
        nrows = (RB - wid + NW - 1) // NW

        def body(i, carry):
            r = wid + i * NW
            pltpu.sync_copy(idx0_hbm.at[r], i0v)
            pltpu.sync_copy(idx1_hbm.at[r], i1v)
            cp0 = pltpu.async_copy(atom_hbm.at[i0v], r0, s0)
            cp1 = pltpu.async_copy(atom_hbm.at[i1v], r1, s1)
            cp0.wait()
            cp1.wait()
            pltpu.sync_copy(r0, out0.at[r])
            pltpu.sync_copy(r1, out1.at[r])
            return carry

        lax.fori_loop(0, nrows, body, 0)

    return gather_attr


def _snorm_body(x_ref, y_ref, z_ref, out_ref):
    x = x_ref[...]
    y = y_ref[...]
    z = z_ref[...]
    r2 = x * x + y * y + z * z
    r = jnp.sqrt(r2)
    r_safe = jnp.maximum(r, 1e-6)
    inv = 1.0 / r_safe
    t = (r - RC) / (RS - RC)
    poly = t * t * t * (10.0 + t * (-15.0 + 6.0 * t)) + 1.0
    mid = inv * poly
    out_ref[...] = jnp.where(r < RS, inv,
                             jnp.where(r < RC, mid, jnp.zeros_like(r)))


def _run_snorm(envx, envy, envz, RB):
    return pl.pallas_call(
        _snorm_body,
        grid=(1,),
        in_specs=[pl.BlockSpec((RB, 128), lambda i: (0, 0))] * 3,
        out_specs=pl.BlockSpec((RB, 128), lambda i: (0, 0)),
        out_shape=jax.ShapeDtypeStruct((RB, 128), F32),
    )(envx, envy, envz)


def _msg_body(sn_ref, env_ref, a0_ref, a1_ref, w1a_ref, w1b_ref, w1c_ref,
              b1_ref, w2_ref, b2_ref, out_ref):
    snorm = sn_ref[...]
    env = env_ref[...]
    a0 = a0_ref[...]
    a1 = a1_ref[...]

    pre = (jnp.dot(snorm, w1a_ref[...], preferred_element_type=F32) +
           jnp.dot(a0, w1b_ref[...], preferred_element_type=F32) +
           jnp.dot(a1, w1c_ref[...], preferred_element_type=F32) +
           b1_ref[...])
    h1 = jnp.tanh(pre)
    h2 = jnp.dot(h1, w2_ref[...], preferred_element_type=F32) + b2_ref[...]

    col10 = lax.broadcasted_iota(I32, (10, 32), 1)
    row10 = lax.broadcasted_iota(I32, (10, 32), 0)
    rmat = jnp.where((col10 < 30) & (col10 // 3 == row10), 1.0, 0.0)
    col3 = lax.broadcasted_iota(I32, (3, 32), 1)
    row3 = lax.broadcasted_iota(I32, (3, 32), 0)
    cmat = jnp.where((col3 < 30) & (col3 % 3 == row3), 1.0, 0.0)
    B = env.shape[0]
    cnt_col = jnp.where(lax.broadcasted_iota(I32, (B, 32), 1) == 30, 1.0, 0.0)
    out = (jnp.dot(h2, rmat, preferred_element_type=F32) *
           jnp.dot(env, cmat, preferred_element_type=F32)) + cnt_col
    out_ref[...] = out


def _run_msg(snorm, env_vectors, attr0_16, attr1_16, W1, b1, W2, b2, E, BE):
    w1a = W1[0:1, :]
    w1b = jnp.zeros((16, 20), F32).at[0:4].set(W1[1:5])
    w1c = jnp.zeros((16, 20), F32).at[0:4].set(W1[5:9])
    grid = (E // BE,)
    return pl.pallas_call(
        _msg_body,
        grid=grid,
        in_specs=[
            pl.BlockSpec((BE, 1), lambda i: (i, 0)),
            pl.BlockSpec((BE, 3), lambda i: (i, 0)),
            pl.BlockSpec((BE, 16), lambda i: (i, 0)),
            pl.BlockSpec((BE, 16), lambda i: (i, 0)),
            pl.BlockSpec((1, 20), lambda i: (0, 0)),
            pl.BlockSpec((16, 20), lambda i: (0, 0)),
            pl.BlockSpec((16, 20), lambda i: (0, 0)),
            pl.BlockSpec((1, 20), lambda i: (0, 0)),
            pl.BlockSpec((20, 10), lambda i: (0, 0)),
            pl.BlockSpec((1, 10), lambda i: (0, 0)),
        ],
        out_specs=pl.BlockSpec((BE, 32), lambda i: (i, 0)),
        out_shape=jax.ShapeDtypeStruct((E, 32), F32),
    )(snorm, env_vectors, attr0_16, attr1_16, w1a, w1b, w1c,
      b1.reshape(1, 20), W2, b2.reshape(1, 10))


def _make_scatter_msg(RB, N):
    RB_SC = RB // NC
    CH = 5
    NCHUNK = RB_SC // CH
    ZR = 125
    NROW_T = N // NS
    NZ = NROW_T // ZR

    @functools.partial(
        pl.kernel,
        out_type=jax.ShapeDtypeStruct((NC, N, 32), F32),
        mesh=_mesh(),
        compiler_params=_SC_PARAMS,
        scratch_types=[
            pltpu.VMEM_SHARED((N, 32), F32),
            pltpu.VMEM((CH, 128, 32), F32),
            pltpu.VMEM((CH, 128), I32),
            pltpu.VMEM((ZR, 32), F32),
        ],
    )
    def scatter_msg(msg_hbm, dst_hbm, out, accum, mbuf, idxbuf, zbuf):
        c = lax.axis_index("c")
        s = lax.axis_index("s")

        zero16 = jnp.zeros((16,), F32)

        def zrow(i, carry):
            zbuf[i, pl.ds(0, 16)] = zero16
            zbuf[i, pl.ds(16, 16)] = zero16
            return carry

        lax.fori_loop(0, ZR, zrow, 0)

        base = s * NROW_T

        def zcopy(k, carry):
            pltpu.sync_copy(zbuf, accum.at[pl.ds(base + k * ZR, ZR)])
            return carry

        lax.fori_loop(0, NZ, zcopy, 0)
        plsc.subcore_barrier()

        nch = (NCHUNK - s + NS - 1) // NS

        def sbody(k, carry):
            j = s + k * NS
            row0 = c * RB_SC + j * CH
            pltpu.sync_copy(msg_hbm.at[pl.ds(row0, CH)], mbuf)
            pltpu.sync_copy(dst_hbm.at[pl.ds(row0, CH)], idxbuf)
            for jj in range(CH):
                pltpu.sync_copy(mbuf.at[jj], accum.at[idxbuf.at[jj]],
                                add=True)
            return carry

        lax.fori_loop(0, nch, sbody, 0)
        plsc.subcore_barrier()

        def obody(k, carry):
            r0 = base + k * ZR
            pltpu.sync_copy(accum.at[pl.ds(r0, ZR)], out.at[c, pl.ds(r0, ZR)])
            return carry

        lax.fori_loop(0, NZ, obody, 0)

    return scatter_msg


def _gram_body(p_ref, out_ref, pad_ref):
    p = p_ref[...]
    sfull = p[0] + p[1]
    cnt = jnp.maximum(sfull[:, 30:31], 1.0)
    a = sfull[:, :30] / cnt

    j30 = lax.broadcasted_iota(I32, (30, 100), 0)
    m = lax.broadcasted_iota(I32, (30, 100), 1)
    out = None
    for c in range(3):
        m1 = jnp.where(j30 == 3 * (m // 10) + c, 1.0, 0.0)
        m2 = jnp.where(j30 == 3 * (m % 10) + c, 1.0, 0.0)
        term = (jnp.dot(a, m1, preferred_element_type=F32) *
                jnp.dot(a, m2, preferred_element_type=F32))
        out = term if out is None else out + term
    out_ref[...] = out
    B = out.shape[0]
    pad_ref[...] = jnp.concatenate([out, jnp.zeros((B, 12), F32)], axis=1)


def _run_gram(partials, N, BN):
    grid = (N // BN,)
    return pl.pallas_call(
        _gram_body,
        grid=grid,
        in_specs=[pl.BlockSpec((2, BN, 32), lambda i: (0, i, 0))],
        out_specs=(
            pl.BlockSpec((BN, 100), lambda i: (i, 0)),
            pl.BlockSpec((BN, 112), lambda i: (i, 0)),
        ),
        out_shape=(
            jax.ShapeDtypeStruct((N, 100), F32),
            jax.ShapeDtypeStruct((N, 112), F32),
        ),
    )(partials)


def _make_edge_gather(RB, N, P):
    @functools.partial(
        pl.kernel,
        out_type=jax.ShapeDtypeStruct((RB, 12800), F32),
        mesh=_mesh(),
        compiler_params=_SC_PARAMS_NL,
        scratch_types=[
            pltpu.VMEM((128,), I32),
            pltpu.VMEM((128,), I32),
            pltpu.VMEM((128, P), F32),
            pltpu.VMEM((128, P), F32),
            pltpu.VMEM((12800,), F32),
            pltpu.SemaphoreType.DMA,
            pltpu.SemaphoreType.DMA,
        ],
    )
    def edge_gather(nd_hbm, idx0_hbm, idx1_hbm, out, i0v, i1v, r0, r1, ob,
                    s0, s1):
        c = lax.axis_index("c")
        s = lax.axis_index("s")
        wid = s * NC + c
        nrows = (RB - wid + NW - 1) // NW
        lanes = lax.iota(I32, 16)

        def body(i, carry):
            r = wid + i * NW
            pltpu.sync_copy(idx0_hbm.at[r], i0v)
            pltpu.sync_copy(idx1_hbm.at[r], i1v)
            cp0 = pltpu.async_copy(nd_hbm.at[i0v], r0, s0)
            cp1 = pltpu.async_copy(nd_hbm.at[i1v], r1, s1)
            cp0.wait()
            cp1.wait()

            @plsc.parallel_loop(0, 800, unroll=8)
            def rep(cc):
                k0 = cc * 16
                kv = k0 + lanes
                iv = lax.shift_right_logical(kv * 41944, 22)
                jv = kv - iv * 100
                g0 = plsc.load_gather(r0, [iv, jv])
                g1 = plsc.load_gather(r1, [iv, jv])
                ob[pl.ds(k0, 16)] = g0 + g1

            pltpu.sync_copy(ob, out.at[r])
            return carry

        lax.fori_loop(0, nrows, body, 0)

    return edge_gather


def kernel(env_vectors, atom_attr, W1, b1, W2, b2, env_index, edge_index):
    N = atom_attr.shape[0]
    E = env_vectors.shape[0]
    A = atom_attr.shape[1]
    RB = E // 128
    P = 112
    BE = 8000
    BN = 2000

    ei0 = env_index[0].reshape(RB, 128)
    ei1 = env_index[1].reshape(RB, 128)
    de0 = edge_index[0].reshape(RB, 128)
    de1 = edge_index[1].reshape(RB, 128)

    atom_pad = jnp.pad(atom_attr, ((0, 0), (0, 16 - A)))
    attr0_3d, attr1_3d = _make_gather_attr(RB, N, 16)(ei0, ei1, atom_pad)
    attr0_16 = attr0_3d.reshape(E, 16)
    attr1_16 = attr1_3d.reshape(E, 16)

    envx = env_vectors[:, 0].reshape(RB, 128)
    envy = env_vectors[:, 1].reshape(RB, 128)
    envz = env_vectors[:, 2].reshape(RB, 128)
    snorm = _run_snorm(envx, envy, envz, RB).reshape(E, 1)

    msg = _run_msg(snorm, env_vectors, attr0_16, attr1_16, W1, b1, W2, b2,
                   E, BE)
    msg3d = msg.reshape(RB, 128, 32)

    partials = _make_scatter_msg(RB, N)(msg3d, ei1)

    node_desc, node_pad = _run_gram(partials, N, BN)

    edge2d = _make_edge_gather(RB, N, P)(node_pad, de0, de1)
    edge_desc = edge2d.reshape(E, 100)

    return node_desc, edge_desc


# speedup vs baseline: 1.2512x; 1.1060x over previous
import functools

import jax
import jax.numpy as jnp
from jax import lax
from jax.experimental import pallas as pl
from jax.experimental.pallas import tpu as pltpu
from jax.experimental.pallas import tpu_sc as plsc

RS = 3.0
RC = 6.0

NC = 2
NS = 16
NW = NC * NS

F32 = jnp.float32
I32 = jnp.int32


def _mesh():
    return plsc.VectorSubcoreMesh(core_axis_name="c", subcore_axis_name="s",
                                  num_cores=NC, num_subcores=NS)


_SC_PARAMS = pltpu.CompilerParams(use_tc_tiling_on_sc=False)
_SC_PARAMS_NL = pltpu.CompilerParams(use_tc_tiling_on_sc=False,
                                     needs_layout_passes=False)


def _make_gather_attr(RB, N, A):
    @functools.partial(
        pl.kernel,
        out_type=(
            jax.ShapeDtypeStruct((RB, 128, A), F32),
            jax.ShapeDtypeStruct((RB, 128, A), F32),
        ),
        mesh=_mesh(),
        compiler_params=_SC_PARAMS,
        scratch_types=[
            pltpu.VMEM((128,), I32),
            pltpu.VMEM((128,), I32),
            pltpu.VMEM((128, A), F32),
            pltpu.VMEM((128, A), F32),
            pltpu.SemaphoreType.DMA,
            pltpu.SemaphoreType.DMA,
        ],
    )
    def gather_attr(idx0_hbm, idx1_hbm, atom_hbm, out0, out1, i0v, i1v, r0, r1,
                    s0, s1):
        c = lax.axis_index("c")
        s = lax.axis_index("s")
        wid = s * NC + c
        nrows = (RB - wid + NW - 1) // NW

        def body(i, carry):
            r = wid + i * NW
            pltpu.sync_copy(idx0_hbm.at[r], i0v)
            pltpu.sync_copy(idx1_hbm.at[r], i1v)
            cp0 = pltpu.async_copy(atom_hbm.at[i0v], r0, s0)
            cp1 = pltpu.async_copy(atom_hbm.at[i1v], r1, s1)
            cp0.wait()
            cp1.wait()
            pltpu.sync_copy(r0, out0.at[r])
            pltpu.sync_copy(r1, out1.at[r])
            return carry

        lax.fori_loop(0, nrows, body, 0)

    return gather_attr


def _snorm_body(x_ref, y_ref, z_ref, out_ref):
    x = x_ref[...]
    y = y_ref[...]
    z = z_ref[...]
    r2 = x * x + y * y + z * z
    r = jnp.sqrt(r2)
    r_safe = jnp.maximum(r, 1e-6)
    inv = 1.0 / r_safe
    t = (r - RC) / (RS - RC)
    poly = t * t * t * (10.0 + t * (-15.0 + 6.0 * t)) + 1.0
    mid = inv * poly
    out_ref[...] = jnp.where(r < RS, inv,
                             jnp.where(r < RC, mid, jnp.zeros_like(r)))


def _run_snorm(envx, envy, envz, RB):
    return pl.pallas_call(
        _snorm_body,
        grid=(1,),
        in_specs=[pl.BlockSpec((RB, 128), lambda i: (0, 0))] * 3,
        out_specs=pl.BlockSpec((RB, 128), lambda i: (0, 0)),
        out_shape=jax.ShapeDtypeStruct((RB, 128), F32),
    )(envx, envy, envz)


def _msg_body(sn_ref, env_ref, a0_ref, a1_ref, w1a_ref, w1b_ref, w1c_ref,
              b1_ref, w2_ref, b2_ref, out_ref):
    snorm = sn_ref[...]
    env = env_ref[...]
    a0 = a0_ref[...]
    a1 = a1_ref[...]

    pre = (jnp.dot(snorm, w1a_ref[...], preferred_element_type=F32) +
           jnp.dot(a0, w1b_ref[...], preferred_element_type=F32) +
           jnp.dot(a1, w1c_ref[...], preferred_element_type=F32) +
           b1_ref[...])
    h1 = jnp.tanh(pre)
    h2 = jnp.dot(h1, w2_ref[...], preferred_element_type=F32) + b2_ref[...]

    col10 = lax.broadcasted_iota(I32, (10, 32), 1)
    row10 = lax.broadcasted_iota(I32, (10, 32), 0)
    rmat = jnp.where((col10 < 30) & (col10 // 3 == row10), 1.0, 0.0)
    col3 = lax.broadcasted_iota(I32, (3, 32), 1)
    row3 = lax.broadcasted_iota(I32, (3, 32), 0)
    cmat = jnp.where((col3 < 30) & (col3 % 3 == row3), 1.0, 0.0)
    B = env.shape[0]
    cnt_col = jnp.where(lax.broadcasted_iota(I32, (B, 32), 1) == 30, 1.0, 0.0)
    out = (jnp.dot(h2, rmat, preferred_element_type=F32) *
           jnp.dot(env, cmat, preferred_element_type=F32)) + cnt_col
    out_ref[...] = out


def _run_msg(snorm, env_vectors, attr0_16, attr1_16, W1, b1, W2, b2, E, BE):
    w1a = W1[0:1, :]
    w1b = jnp.zeros((16, 20), F32).at[0:4].set(W1[1:5])
    w1c = jnp.zeros((16, 20), F32).at[0:4].set(W1[5:9])
    grid = (E // BE,)
    return pl.pallas_call(
        _msg_body,
        grid=grid,
        in_specs=[
            pl.BlockSpec((BE, 1), lambda i: (i, 0)),
            pl.BlockSpec((BE, 3), lambda i: (i, 0)),
            pl.BlockSpec((BE, 16), lambda i: (i, 0)),
            pl.BlockSpec((BE, 16), lambda i: (i, 0)),
            pl.BlockSpec((1, 20), lambda i: (0, 0)),
            pl.BlockSpec((16, 20), lambda i: (0, 0)),
            pl.BlockSpec((16, 20), lambda i: (0, 0)),
            pl.BlockSpec((1, 20), lambda i: (0, 0)),
            pl.BlockSpec((20, 10), lambda i: (0, 0)),
            pl.BlockSpec((1, 10), lambda i: (0, 0)),
        ],
        out_specs=pl.BlockSpec((BE, 32), lambda i: (i, 0)),
        out_shape=jax.ShapeDtypeStruct((E, 32), F32),
    )(snorm, env_vectors, attr0_16, attr1_16, w1a, w1b, w1c,
      b1.reshape(1, 20), W2, b2.reshape(1, 10))


def _make_scatter_msg(RB, N):
    RB_SC = RB // NC
    CH = 5
    NCHUNK = RB_SC // CH
    ZR = 125
    NROW_T = N // NS
    NZ = NROW_T // ZR

    @functools.partial(
        pl.kernel,
        out_type=jax.ShapeDtypeStruct((NC, N, 32), F32),
        mesh=_mesh(),
        compiler_params=_SC_PARAMS,
        scratch_types=[
            pltpu.VMEM_SHARED((N, 32), F32),
            pltpu.VMEM((CH, 128, 32), F32),
            pltpu.VMEM((CH, 128), I32),
            pltpu.VMEM((ZR, 32), F32),
        ],
    )
    def scatter_msg(msg_hbm, dst_hbm, out, accum, mbuf, idxbuf, zbuf):
        c = lax.axis_index("c")
        s = lax.axis_index("s")

        zero16 = jnp.zeros((16,), F32)

        def zrow(i, carry):
            zbuf[i, pl.ds(0, 16)] = zero16
            zbuf[i, pl.ds(16, 16)] = zero16
            return carry

        lax.fori_loop(0, ZR, zrow, 0)

        base = s * NROW_T

        def zcopy(k, carry):
            pltpu.sync_copy(zbuf, accum.at[pl.ds(base + k * ZR, ZR)])
            return carry

        lax.fori_loop(0, NZ, zcopy, 0)
        plsc.subcore_barrier()

        nch = (NCHUNK - s + NS - 1) // NS

        def sbody(k, carry):
            j = s + k * NS
            row0 = c * RB_SC + j * CH
            pltpu.sync_copy(msg_hbm.at[pl.ds(row0, CH)], mbuf)
            pltpu.sync_copy(dst_hbm.at[pl.ds(row0, CH)], idxbuf)
            for jj in range(CH):
                pltpu.sync_copy(mbuf.at[jj], accum.at[idxbuf.at[jj]],
                                add=True)
            return carry

        lax.fori_loop(0, nch, sbody, 0)
        plsc.subcore_barrier()

        def obody(k, carry):
            r0 = base + k * ZR
            pltpu.sync_copy(accum.at[pl.ds(r0, ZR)], out.at[c, pl.ds(r0, ZR)])
            return carry

        lax.fori_loop(0, NZ, obody, 0)

    return scatter_msg


def _gram_body(p_ref, out_ref, pad_ref):
    p = p_ref[...]
    sfull = p[0] + p[1]
    cnt = jnp.maximum(sfull[:, 30:31], 1.0)
    a = sfull[:, :30] / cnt

    j30 = lax.broadcasted_iota(I32, (30, 100), 0)
    m = lax.broadcasted_iota(I32, (30, 100), 1)
    out = None
    for c in range(3):
        m1 = jnp.where(j30 == 3 * (m // 10) + c, 1.0, 0.0)
        m2 = jnp.where(j30 == 3 * (m % 10) + c, 1.0, 0.0)
        term = (jnp.dot(a, m1, preferred_element_type=F32) *
                jnp.dot(a, m2, preferred_element_type=F32))
        out = term if out is None else out + term
    out_ref[...] = out
    B = out.shape[0]
    pad_ref[...] = jnp.concatenate([out, jnp.zeros((B, 12), F32)], axis=1)


def _run_gram(partials, N, BN):
    grid = (N // BN,)
    return pl.pallas_call(
        _gram_body,
        grid=grid,
        in_specs=[pl.BlockSpec((2, BN, 32), lambda i: (0, i, 0))],
        out_specs=(
            pl.BlockSpec((BN, 100), lambda i: (i, 0)),
            pl.BlockSpec((BN, 112), lambda i: (i, 0)),
        ),
        out_shape=(
            jax.ShapeDtypeStruct((N, 100), F32),
            jax.ShapeDtypeStruct((N, 112), F32),
        ),
    )(partials)


def _make_edge_gather(RB, N, P):
    @functools.partial(
        pl.kernel,
        out_type=jax.ShapeDtypeStruct((RB, 12800), F32),
        mesh=_mesh(),
        compiler_params=_SC_PARAMS_NL,
        scratch_types=[
            pltpu.VMEM((2, 128), I32),
            pltpu.VMEM((2, 128), I32),
            pltpu.VMEM((2, 128, P), F32),
            pltpu.VMEM((2, 128, P), F32),
            pltpu.VMEM((2, 12800), F32),
            pltpu.SemaphoreType.DMA((2,)),
            pltpu.SemaphoreType.DMA((2,)),
        ],
    )
    def edge_gather(nd_hbm, idx0_hbm, idx1_hbm, out, i0v, i1v, r0, r1, ob,
                    sg, so):
        c = lax.axis_index("c")
        s = lax.axis_index("s")
        wid = s * NC + c
        nrows = (RB - wid + NW - 1) // NW
        lanes = lax.iota(I32, 16)

        def issue(blk, p):
            r = wid + blk * NW
            pltpu.sync_copy(idx0_hbm.at[r], i0v.at[p])
            pltpu.sync_copy(idx1_hbm.at[r], i1v.at[p])
            pltpu.async_copy(nd_hbm.at[i0v.at[p]], r0.at[p], sg.at[p])
            pltpu.async_copy(nd_hbm.at[i1v.at[p]], r1.at[p], sg.at[p])

        issue(0, 0)

        def body(i, carry):
            p = lax.rem(i, 2)
            q = 1 - p

            @pl.when(i + 1 < nrows)
            def _():
                issue(i + 1, q)

            pltpu.make_async_copy(nd_hbm.at[i0v.at[p]], r0.at[p],
                                  sg.at[p]).wait()
            pltpu.make_async_copy(nd_hbm.at[i1v.at[p]], r1.at[p],
                                  sg.at[p]).wait()

            @pl.when(i >= 2)
            def _():
                pltpu.make_async_copy(out.at[0], ob.at[p], so.at[p]).wait()

            r0p = r0.at[p]
            r1p = r1.at[p]
            obp = ob.at[p]

            @plsc.parallel_loop(0, 800, unroll=8)
            def rep(cc):
                k0 = cc * 16
                kv = k0 + lanes
                iv = lax.shift_right_logical(kv * 41944, 22)
                jv = kv - iv * 100
                g0 = plsc.load_gather(r0p, [iv, jv])
                g1 = plsc.load_gather(r1p, [iv, jv])
                obp[pl.ds(k0, 16)] = g0 + g1

            r = wid + i * NW
            pltpu.async_copy(ob.at[p], out.at[r], so.at[p])
            return carry

        lax.fori_loop(0, nrows, body, 0)
        pltpu.make_async_copy(out.at[0], ob.at[0], so.at[0]).wait()
        pltpu.make_async_copy(out.at[0], ob.at[1], so.at[1]).wait()

    return edge_gather


def kernel(env_vectors, atom_attr, W1, b1, W2, b2, env_index, edge_index):
    N = atom_attr.shape[0]
    E = env_vectors.shape[0]
    A = atom_attr.shape[1]
    RB = E // 128
    P = 112
    BE = 8000
    BN = 2000

    ei0 = env_index[0].reshape(RB, 128)
    ei1 = env_index[1].reshape(RB, 128)
    de0 = edge_index[0].reshape(RB, 128)
    de1 = edge_index[1].reshape(RB, 128)

    atom_pad = jnp.pad(atom_attr, ((0, 0), (0, 16 - A)))
    attr0_3d, attr1_3d = _make_gather_attr(RB, N, 16)(ei0, ei1, atom_pad)
    attr0_16 = attr0_3d.reshape(E, 16)
    attr1_16 = attr1_3d.reshape(E, 16)

    envx = env_vectors[:, 0].reshape(RB, 128)
    envy = env_vectors[:, 1].reshape(RB, 128)
    envz = env_vectors[:, 2].reshape(RB, 128)
    snorm = _run_snorm(envx, envy, envz, RB).reshape(E, 1)

    msg = _run_msg(snorm, env_vectors, attr0_16, attr1_16, W1, b1, W2, b2,
                   E, BE)
    msg3d = msg.reshape(RB, 128, 32)

    partials = _make_scatter_msg(RB, N)(msg3d, ei1)

    node_desc, node_pad = _run_gram(partials, N, BN)

    edge2d = _make_edge_gather(RB, N, P)(node_pad, de0, de1)
    edge_desc = edge2d.reshape(E, 100)

    return node_desc, edge_desc


# speedup vs baseline: 1.2790x; 1.0222x over previous
import functools

import jax
import jax.numpy as jnp
from jax import lax
from jax.experimental import pallas as pl
from jax.experimental.pallas import tpu as pltpu
from jax.experimental.pallas import tpu_sc as plsc

RS = 3.0
RC = 6.0

NC = 2
NS = 16
NW = NC * NS

F32 = jnp.float32
I32 = jnp.int32


def _mesh():
    return plsc.VectorSubcoreMesh(core_axis_name="c", subcore_axis_name="s",
                                  num_cores=NC, num_subcores=NS)


_SC_PARAMS = pltpu.CompilerParams(use_tc_tiling_on_sc=False)
_SC_PARAMS_NL = pltpu.CompilerParams(use_tc_tiling_on_sc=False,
                                     needs_layout_passes=False)


def _make_gather_attr(RB, N, A):
    @functools.partial(
        pl.kernel,
        out_type=(
            jax.ShapeDtypeStruct((RB, 128, A), F32),
            jax.ShapeDtypeStruct((RB, 128, A), F32),
        ),
        mesh=_mesh(),
        compiler_params=_SC_PARAMS,
        scratch_types=[
            pltpu.VMEM((2, 128), I32),
            pltpu.VMEM((2, 128), I32),
            pltpu.VMEM((2, 128, A), F32),
            pltpu.VMEM((2, 128, A), F32),
            pltpu.SemaphoreType.DMA((2,)),
            pltpu.SemaphoreType.DMA((2,)),
            pltpu.SemaphoreType.DMA((2,)),
        ],
    )
    def gather_attr(idx0_hbm, idx1_hbm, atom_hbm, out0, out1, i0v, i1v, r0, r1,
                    sg, so0, so1):
        c = lax.axis_index("c")
        s = lax.axis_index("s")
        wid = s * NC + c
        nrows = (RB - wid + NW - 1) // NW

        def issue(blk, p):
            r = wid + blk * NW
            pltpu.sync_copy(idx0_hbm.at[r], i0v.at[p])
            pltpu.sync_copy(idx1_hbm.at[r], i1v.at[p])
            pltpu.async_copy(atom_hbm.at[i0v.at[p]], r0.at[p], sg.at[p])
            pltpu.async_copy(atom_hbm.at[i1v.at[p]], r1.at[p], sg.at[p])

        issue(0, 0)

        def body(i, carry):
            p = lax.rem(i, 2)
            q = 1 - p

            @pl.when(i >= 1)
            def _():
                pltpu.make_async_copy(out0.at[0], r0.at[q], so0.at[q]).wait()
                pltpu.make_async_copy(out1.at[0], r1.at[q], so1.at[q]).wait()

            @pl.when(i + 1 < nrows)
            def _():
                issue(i + 1, q)

            pltpu.make_async_copy(atom_hbm.at[i0v.at[p]], r0.at[p],
                                  sg.at[p]).wait()
            pltpu.make_async_copy(atom_hbm.at[i1v.at[p]], r1.at[p],
                                  sg.at[p]).wait()

            r = wid + i * NW
            pltpu.async_copy(r0.at[p], out0.at[r], so0.at[p])
            pltpu.async_copy(r1.at[p], out1.at[r], so1.at[p])
            return carry

        lax.fori_loop(0, nrows, body, 0)
        pf = lax.rem(nrows - 1, 2)
        pltpu.make_async_copy(out0.at[0], r0.at[pf], so0.at[pf]).wait()
        pltpu.make_async_copy(out1.at[0], r1.at[pf], so1.at[pf]).wait()

    return gather_attr


def _snorm_body(x_ref, y_ref, z_ref, out_ref):
    x = x_ref[...]
    y = y_ref[...]
    z = z_ref[...]
    r2 = x * x + y * y + z * z
    r = jnp.sqrt(r2)
    r_safe = jnp.maximum(r, 1e-6)
    inv = 1.0 / r_safe
    t = (r - RC) / (RS - RC)
    poly = t * t * t * (10.0 + t * (-15.0 + 6.0 * t)) + 1.0
    mid = inv * poly
    out_ref[...] = jnp.where(r < RS, inv,
                             jnp.where(r < RC, mid, jnp.zeros_like(r)))


def _run_snorm(envx, envy, envz, RB):
    return pl.pallas_call(
        _snorm_body,
        grid=(1,),
        in_specs=[pl.BlockSpec((RB, 128), lambda i: (0, 0))] * 3,
        out_specs=pl.BlockSpec((RB, 128), lambda i: (0, 0)),
        out_shape=jax.ShapeDtypeStruct((RB, 128), F32),
    )(envx, envy, envz)


def _msg_body(sn_ref, env_ref, a0_ref, a1_ref, w1a_ref, w1b_ref, w1c_ref,
              b1_ref, w2_ref, b2_ref, out_ref):
    snorm = sn_ref[...]
    env = env_ref[...]
    a0 = a0_ref[...]
    a1 = a1_ref[...]

    pre = (jnp.dot(snorm, w1a_ref[...], preferred_element_type=F32) +
           jnp.dot(a0, w1b_ref[...], preferred_element_type=F32) +
           jnp.dot(a1, w1c_ref[...], preferred_element_type=F32) +
           b1_ref[...])
    h1 = jnp.tanh(pre)
    h2 = jnp.dot(h1, w2_ref[...], preferred_element_type=F32) + b2_ref[...]

    col10 = lax.broadcasted_iota(I32, (10, 32), 1)
    row10 = lax.broadcasted_iota(I32, (10, 32), 0)
    rmat = jnp.where((col10 < 30) & (col10 // 3 == row10), 1.0, 0.0)
    col3 = lax.broadcasted_iota(I32, (3, 32), 1)
    row3 = lax.broadcasted_iota(I32, (3, 32), 0)
    cmat = jnp.where((col3 < 30) & (col3 % 3 == row3), 1.0, 0.0)
    B = env.shape[0]
    cnt_col = jnp.where(lax.broadcasted_iota(I32, (B, 32), 1) == 30, 1.0, 0.0)
    out = (jnp.dot(h2, rmat, preferred_element_type=F32) *
           jnp.dot(env, cmat, preferred_element_type=F32)) + cnt_col
    out_ref[...] = out


def _run_msg(snorm, env_vectors, attr0_16, attr1_16, W1, b1, W2, b2, E, BE):
    w1a = W1[0:1, :]
    w1b = jnp.zeros((16, 20), F32).at[0:4].set(W1[1:5])
    w1c = jnp.zeros((16, 20), F32).at[0:4].set(W1[5:9])
    grid = (E // BE,)
    return pl.pallas_call(
        _msg_body,
        grid=grid,
        in_specs=[
            pl.BlockSpec((BE, 1), lambda i: (i, 0)),
            pl.BlockSpec((BE, 3), lambda i: (i, 0)),
            pl.BlockSpec((BE, 16), lambda i: (i, 0)),
            pl.BlockSpec((BE, 16), lambda i: (i, 0)),
            pl.BlockSpec((1, 20), lambda i: (0, 0)),
            pl.BlockSpec((16, 20), lambda i: (0, 0)),
            pl.BlockSpec((16, 20), lambda i: (0, 0)),
            pl.BlockSpec((1, 20), lambda i: (0, 0)),
            pl.BlockSpec((20, 10), lambda i: (0, 0)),
            pl.BlockSpec((1, 10), lambda i: (0, 0)),
        ],
        out_specs=pl.BlockSpec((BE, 32), lambda i: (i, 0)),
        out_shape=jax.ShapeDtypeStruct((E, 32), F32),
    )(snorm, env_vectors, attr0_16, attr1_16, w1a, w1b, w1c,
      b1.reshape(1, 20), W2, b2.reshape(1, 10))


def _make_scatter_msg(RB, N):
    RB_SC = RB // NC
    CH = 5
    NCHUNK = RB_SC // CH
    ZR = 125
    NROW_T = N // NS
    NZ = NROW_T // ZR

    @functools.partial(
        pl.kernel,
        out_type=jax.ShapeDtypeStruct((NC, N, 32), F32),
        mesh=_mesh(),
        compiler_params=_SC_PARAMS,
        scratch_types=[
            pltpu.VMEM_SHARED((N, 32), F32),
            pltpu.VMEM((CH, 128, 32), F32),
            pltpu.VMEM((CH, 128), I32),
            pltpu.VMEM((ZR, 32), F32),
        ],
    )
    def scatter_msg(msg_hbm, dst_hbm, out, accum, mbuf, idxbuf, zbuf):
        c = lax.axis_index("c")
        s = lax.axis_index("s")

        zero16 = jnp.zeros((16,), F32)

        def zrow(i, carry):
            zbuf[i, pl.ds(0, 16)] = zero16
            zbuf[i, pl.ds(16, 16)] = zero16
            return carry

        lax.fori_loop(0, ZR, zrow, 0)

        base = s * NROW_T

        def zcopy(k, carry):
            pltpu.sync_copy(zbuf, accum.at[pl.ds(base + k * ZR, ZR)])
            return carry

        lax.fori_loop(0, NZ, zcopy, 0)
        plsc.subcore_barrier()

        nch = (NCHUNK - s + NS - 1) // NS

        def sbody(k, carry):
            j = s + k * NS
            row0 = c * RB_SC + j * CH
            pltpu.sync_copy(msg_hbm.at[pl.ds(row0, CH)], mbuf)
            pltpu.sync_copy(dst_hbm.at[pl.ds(row0, CH)], idxbuf)
            for jj in range(CH):
                pltpu.sync_copy(mbuf.at[jj], accum.at[idxbuf.at[jj]],
                                add=True)
            return carry

        lax.fori_loop(0, nch, sbody, 0)
        plsc.subcore_barrier()

        def obody(k, carry):
            r0 = base + k * ZR
            pltpu.sync_copy(accum.at[pl.ds(r0, ZR)], out.at[c, pl.ds(r0, ZR)])
            return carry

        lax.fori_loop(0, NZ, obody, 0)

    return scatter_msg


def _gram_body(p_ref, out_ref, pad_ref):
    p = p_ref[...]
    sfull = p[0] + p[1]
    cnt = jnp.maximum(sfull[:, 30:31], 1.0)
    a = sfull[:, :30] / cnt

    j30 = lax.broadcasted_iota(I32, (30, 100), 0)
    m = lax.broadcasted_iota(I32, (30, 100), 1)
    out = None
    for c in range(3):
        m1 = jnp.where(j30 == 3 * (m // 10) + c, 1.0, 0.0)
        m2 = jnp.where(j30 == 3 * (m % 10) + c, 1.0, 0.0)
        term = (jnp.dot(a, m1, preferred_element_type=F32) *
                jnp.dot(a, m2, preferred_element_type=F32))
        out = term if out is None else out + term
    out_ref[...] = out
    B = out.shape[0]
    pad_ref[...] = jnp.concatenate([out, jnp.zeros((B, 12), F32)], axis=1)


def _run_gram(partials, N, BN):
    grid = (N // BN,)
    return pl.pallas_call(
        _gram_body,
        grid=grid,
        in_specs=[pl.BlockSpec((2, BN, 32), lambda i: (0, i, 0))],
        out_specs=(
            pl.BlockSpec((BN, 100), lambda i: (i, 0)),
            pl.BlockSpec((BN, 112), lambda i: (i, 0)),
        ),
        out_shape=(
            jax.ShapeDtypeStruct((N, 100), F32),
            jax.ShapeDtypeStruct((N, 112), F32),
        ),
    )(partials)


def _make_edge_gather(RB, N, P):
    @functools.partial(
        pl.kernel,
        out_type=jax.ShapeDtypeStruct((RB, 12800), F32),
        mesh=_mesh(),
        compiler_params=_SC_PARAMS_NL,
        scratch_types=[
            pltpu.VMEM((2, 128), I32),
            pltpu.VMEM((2, 128), I32),
            pltpu.VMEM((2, 128, P), F32),
            pltpu.VMEM((2, 128, P), F32),
            pltpu.VMEM((2, 12800), F32),
            pltpu.SemaphoreType.DMA((2,)),
            pltpu.SemaphoreType.DMA((2,)),
        ],
    )
    def edge_gather(nd_hbm, idx0_hbm, idx1_hbm, out, i0v, i1v, r0, r1, ob,
                    sg, so):
        c = lax.axis_index("c")
        s = lax.axis_index("s")
        wid = s * NC + c
        nrows = (RB - wid + NW - 1) // NW
        lanes = lax.iota(I32, 16)

        def issue(blk, p):
            r = wid + blk * NW
            pltpu.sync_copy(idx0_hbm.at[r], i0v.at[p])
            pltpu.sync_copy(idx1_hbm.at[r], i1v.at[p])
            pltpu.async_copy(nd_hbm.at[i0v.at[p]], r0.at[p], sg.at[p])
            pltpu.async_copy(nd_hbm.at[i1v.at[p]], r1.at[p], sg.at[p])

        issue(0, 0)

        def body(i, carry):
            p = lax.rem(i, 2)
            q = 1 - p

            @pl.when(i + 1 < nrows)
            def _():
                issue(i + 1, q)

            pltpu.make_async_copy(nd_hbm.at[i0v.at[p]], r0.at[p],
                                  sg.at[p]).wait()
            pltpu.make_async_copy(nd_hbm.at[i1v.at[p]], r1.at[p],
                                  sg.at[p]).wait()

            @pl.when(i >= 2)
            def _():
                pltpu.make_async_copy(out.at[0], ob.at[p], so.at[p]).wait()

            r0p = r0.at[p]
            r1p = r1.at[p]
            obp = ob.at[p]

            @plsc.parallel_loop(0, 800, unroll=8)
            def rep(cc):
                k0 = cc * 16
                kv = k0 + lanes
                iv = lax.shift_right_logical(kv * 41944, 22)
                jv = kv - iv * 100
                g0 = plsc.load_gather(r0p, [iv, jv])
                g1 = plsc.load_gather(r1p, [iv, jv])
                obp[pl.ds(k0, 16)] = g0 + g1

            r = wid + i * NW
            pltpu.async_copy(ob.at[p], out.at[r], so.at[p])
            return carry

        lax.fori_loop(0, nrows, body, 0)
        pltpu.make_async_copy(out.at[0], ob.at[0], so.at[0]).wait()
        pltpu.make_async_copy(out.at[0], ob.at[1], so.at[1]).wait()

    return edge_gather


def kernel(env_vectors, atom_attr, W1, b1, W2, b2, env_index, edge_index):
    N = atom_attr.shape[0]
    E = env_vectors.shape[0]
    A = atom_attr.shape[1]
    RB = E // 128
    P = 112
    BE = 8000
    BN = 2000

    ei0 = env_index[0].reshape(RB, 128)
    ei1 = env_index[1].reshape(RB, 128)
    de0 = edge_index[0].reshape(RB, 128)
    de1 = edge_index[1].reshape(RB, 128)

    atom_pad = jnp.pad(atom_attr, ((0, 0), (0, 16 - A)))
    attr0_3d, attr1_3d = _make_gather_attr(RB, N, 16)(ei0, ei1, atom_pad)
    attr0_16 = attr0_3d.reshape(E, 16)
    attr1_16 = attr1_3d.reshape(E, 16)

    envx = env_vectors[:, 0].reshape(RB, 128)
    envy = env_vectors[:, 1].reshape(RB, 128)
    envz = env_vectors[:, 2].reshape(RB, 128)
    snorm = _run_snorm(envx, envy, envz, RB).reshape(E, 1)

    msg = _run_msg(snorm, env_vectors, attr0_16, attr1_16, W1, b1, W2, b2,
                   E, BE)
    msg3d = msg.reshape(RB, 128, 32)

    partials = _make_scatter_msg(RB, N)(msg3d, ei1)

    node_desc, node_pad = _run_gram(partials, N, BN)

    edge2d = _make_edge_gather(RB, N, P)(node_pad, de0, de1)
    edge_desc = edge2d.reshape(E, 100)

    return node_desc, edge_desc
